# baseline (speedup 1.0000x reference)
import functools

import jax
import jax.numpy as jnp
import numpy as np
from jax.experimental import pallas as pl
from jax.experimental.pallas import tpu as pltpu

_B = 2
_N = 8192
_NUM_IMAGES = 3
_IMG_H, _IMG_W = 256, 320
_FEAT_H, _FEAT_W = 32, 40
_P_PROJ = 4096
_NUM_CLASSES = 21


def _batchnorm(x, axes):
    mean = jnp.mean(x, axis=axes, keepdims=True)
    var = jnp.var(x, axis=axes, keepdims=True)
    return (x - mean) / jnp.sqrt(var + 1e-5)


def _square_distance(a, b):
    return (jnp.sum(a * a, -1)[:, :, None] + jnp.sum(b * b, -1)[:, None, :]
            - 2.0 * jnp.einsum('bnc,bmc->bnm', a, b))


def _index_points(pts, idx):
    return jax.vmap(lambda p, i: p[i])(pts, idx)


def _fps(xyz, npoint):
    xyz = jax.lax.stop_gradient(xyz)
    Bc, Nc, _ = xyz.shape

    def body(i, state):
        centroids, distance, farthest = state
        centroids = centroids.at[:, i].set(farthest)
        centroid = jax.vmap(lambda p, f: p[f])(xyz, farthest)
        dist = jnp.sum((xyz - centroid[:, None, :]) ** 2, -1)
        distance = jnp.minimum(distance, dist)
        farthest = jnp.argmax(distance, axis=-1).astype(jnp.int32)
        return centroids, distance, farthest

    init = (jnp.zeros((Bc, npoint), jnp.int32),
            jnp.full((Bc, Nc), 1e10, jnp.float32),
            jnp.zeros((Bc,), jnp.int32))
    centroids, _, _ = jax.lax.fori_loop(0, npoint, body, init)
    return centroids


def _query_ball(radius, nsample, xyz, new_xyz):
    Bc, S, _ = new_xyz.shape
    Nc = xyz.shape[1]
    sqrdists = jax.lax.stop_gradient(_square_distance(new_xyz, xyz))
    group_idx = jnp.broadcast_to(jnp.arange(Nc, dtype=jnp.int32), (Bc, S, Nc))
    group_idx = jnp.where(sqrdists > radius ** 2, Nc, group_idx)
    group_idx = jnp.sort(group_idx, axis=-1)[:, :, :nsample]
    group_first = jnp.broadcast_to(group_idx[:, :, :1], group_idx.shape)
    group_idx = jnp.where(group_idx == Nc, group_first, group_idx)
    return group_idx


def _sa_layer(ps, npoint, radius, nsample, xyz, points):
    xyz_t = jnp.transpose(xyz, (0, 2, 1))
    points_t = jnp.transpose(points, (0, 2, 1))
    fps_idx = _fps(xyz_t, npoint)
    new_xyz = _index_points(xyz_t, fps_idx)
    idx = _query_ball(radius, nsample, xyz_t, new_xyz)
    grouped_xyz = _index_points(xyz_t, idx) - new_xyz[:, :, None, :]
    grouped_points = _index_points(points_t, idx)
    new_points = jnp.concatenate([grouped_xyz, grouped_points], axis=-1)
    x = jnp.transpose(new_points, (0, 3, 2, 1))
    for layer in ps:
        x = jnp.einsum('oc,bcns->bons', layer["W"], x) + layer["b"][None, :, None, None]
        x = jax.nn.relu(_batchnorm(x, (0, 2, 3)))
    x = jnp.max(x, axis=2)
    return jnp.transpose(new_xyz, (0, 2, 1)), x


def _fp_layer(ps, xyz1, xyz2, points1, points2):
    xyz1_t = jnp.transpose(xyz1, (0, 2, 1))
    xyz2_t = jnp.transpose(xyz2, (0, 2, 1))
    points2_t = jnp.transpose(points2, (0, 2, 1))
    dists = _square_distance(xyz1_t, xyz2_t)
    idx = jnp.argsort(dists, axis=-1)[:, :, :3]
    d = jnp.take_along_axis(dists, idx, axis=-1)
    dist_recip = 1.0 / (d + 1e-8)
    norm = jnp.sum(dist_recip, axis=2, keepdims=True)
    weight = dist_recip / norm
    interpolated = jnp.sum(_index_points(points2_t, idx) * weight[..., None], axis=2)
    if points1 is not None:
        points1_t = jnp.transpose(points1, (0, 2, 1))
        new_points = jnp.concatenate([points1_t, interpolated], axis=-1)
    else:
        new_points = interpolated
    x = jnp.transpose(new_points, (0, 2, 1))
    for layer in ps:
        x = jnp.einsum('oc,bcn->bon', layer["W"], x) + layer["b"][None, :, None]
        x = jax.nn.relu(_batchnorm(x, (0, 2)))
    return x


def _enet_features(ps, imgs):
    dn = ('NCHW', 'OIHW', 'NCHW')
    x = jax.lax.conv_general_dilated(imgs, ps["W1"], (4, 4), 'VALID', dimension_numbers=dn)
    x = jax.nn.relu(x + ps["b1"][None, :, None, None])
    x = jax.lax.conv_general_dilated(x, ps["W2"], (2, 2), 'VALID', dimension_numbers=dn)
    x = jax.nn.relu(x + ps["b2"][None, :, None, None])
    return x


def _projection(ft, ind3d, ind2d, num_points):
    C = ft.shape[0]
    flat = ft.reshape(C, -1)
    gathered = flat[:, ind2d]
    return jnp.zeros((C, num_points), ft.dtype).at[:, ind3d].set(gathered)


def _head_kernel(x_ref, w1_ref, b1_ref, w2_ref, b2_ref, o_ref):
    x = x_ref[...]
    y = jnp.dot(w1_ref[...], x, preferred_element_type=jnp.float32)
    y = y + b1_ref[...].reshape(-1, 1)
    mean = jnp.mean(y, axis=1, keepdims=True)
    var = jnp.mean((y - mean) ** 2, axis=1, keepdims=True)
    y = jax.nn.relu((y - mean) / jnp.sqrt(var + 1e-5))
    z = jnp.dot(w2_ref[...], y, preferred_element_type=jnp.float32)
    o_ref[...] = z + b2_ref[...].reshape(-1, 1)


def _head(params, l0_points):
    Bc, C, Nc = l0_points.shape
    x = jnp.transpose(l0_points, (1, 0, 2)).reshape(C, Bc * Nc)
    out = pl.pallas_call(
        _head_kernel,
        out_shape=jax.ShapeDtypeStruct((_NUM_CLASSES, Bc * Nc), jnp.float32),
    )(x, params["conv1"]["W"], params["conv1"]["b"],
      params["conv2"]["W"], params["conv2"]["b"])
    return jnp.transpose(out.reshape(_NUM_CLASSES, Bc, Nc), (1, 2, 0))


def kernel(params, xyz, points, image, projection_indices_3d, projection_indices_2d):
    num_points = xyz.shape[2]
    feats = []
    for i in range(_B):
        ft = _enet_features(params["enet"], image[i])
        proj = [_projection(ft[j], projection_indices_3d[i, j],
                            projection_indices_2d[i, j], num_points)
                for j in range(_NUM_IMAGES)]
        imageft = jnp.stack(proj, axis=2)
        imageft = jnp.max(imageft, axis=2)
        feats.append(imageft)
    image_features = jnp.stack(feats, axis=0)
    l1_xyz, l1_points = _sa_layer(params["sa1_geo"], 1024, 0.1, 32, xyz, points)
    l2_xyz, l2_points = _sa_layer(params["sa2_geo"], 256, 0.2, 32, l1_xyz, l1_points)
    l1_xyz_f, l1_points_f = _sa_layer(params["sa1_feat"], 1024, 0.1, 32, xyz, image_features)
    l2_xyz_f, l2_points_f = _sa_layer(params["sa2_feat"], 256, 0.2, 32, l1_xyz_f, l1_points_f)
    l2_points = jnp.concatenate([l2_points, l2_points_f], axis=1)
    l3_xyz, l3_points = _sa_layer(params["sa3"], 64, 0.4, 32, l2_xyz, l2_points)
    l4_xyz, l4_points = _sa_layer(params["sa4"], 16, 0.8, 32, l3_xyz, l3_points)
    l3_points = _fp_layer(params["fp4"], l3_xyz, l4_xyz, l3_points, l4_points)
    l2_points = _fp_layer(params["fp3"], l2_xyz, l3_xyz, l2_points, l3_points)
    l1_points = _fp_layer(params["fp2"], l1_xyz, l2_xyz, l1_points, l2_points)
    l0_points = _fp_layer(params["fp1"], xyz, l1_xyz, points, l1_points)
    return _head(params, l0_points)


# speedup vs baseline: 1.4583x; 1.4583x over previous
import functools

import jax
import jax.numpy as jnp
import numpy as np
from jax.experimental import pallas as pl
from jax.experimental.pallas import tpu as pltpu

_B = 2
_N = 8192
_NUM_IMAGES = 3
_IMG_H, _IMG_W = 256, 320
_FEAT_H, _FEAT_W = 32, 40
_P_PROJ = 4096
_NUM_CLASSES = 21


def _batchnorm(x, axes):
    mean = jnp.mean(x, axis=axes, keepdims=True)
    var = jnp.var(x, axis=axes, keepdims=True)
    return (x - mean) / jnp.sqrt(var + 1e-5)


def _square_distance(a, b):
    return (jnp.sum(a * a, -1)[:, :, None] + jnp.sum(b * b, -1)[:, None, :]
            - 2.0 * jnp.einsum('bnc,bmc->bnm', a, b))


def _index_points(pts, idx):
    return jax.vmap(lambda p, i: p[i])(pts, idx)


def _fps(xyz, npoint):
    xyz = jax.lax.stop_gradient(xyz)
    Bc, Nc, _ = xyz.shape

    def body(i, state):
        centroids, distance, farthest = state
        centroids = centroids.at[:, i].set(farthest)
        centroid = jax.vmap(lambda p, f: p[f])(xyz, farthest)
        dist = jnp.sum((xyz - centroid[:, None, :]) ** 2, -1)
        distance = jnp.minimum(distance, dist)
        farthest = jnp.argmax(distance, axis=-1).astype(jnp.int32)
        return centroids, distance, farthest

    init = (jnp.zeros((Bc, npoint), jnp.int32),
            jnp.full((Bc, Nc), 1e10, jnp.float32),
            jnp.zeros((Bc,), jnp.int32))
    centroids, _, _ = jax.lax.fori_loop(0, npoint, body, init)
    return centroids


def _bq_kernel(d2_ref, o_ref, *, r2, K, Nc):
    d2 = d2_ref[0]
    iota = jax.lax.broadcasted_iota(jnp.int32, d2.shape, 1)
    m = jnp.where(d2 <= r2, iota, Nc)
    cols = []
    for _ in range(K):
        cur = jnp.min(m, axis=1, keepdims=True)
        cols.append(cur)
        m = jnp.where(m == cur, Nc, m)
    idx = jnp.concatenate(cols, axis=1)
    first = idx[:, 0:1]
    o_ref[0] = jnp.where(idx == Nc, jnp.broadcast_to(first, idx.shape), idx)


def _query_ball_pallas(radius, nsample, xyz_t, new_xyz):
    Bc, S, _ = new_xyz.shape
    Nc = xyz_t.shape[1]
    d2 = jax.lax.stop_gradient(_square_distance(new_xyz, xyz_t))
    BS = min(S, 256)
    kfn = functools.partial(_bq_kernel, r2=radius ** 2, K=nsample, Nc=Nc)
    return pl.pallas_call(
        kfn,
        grid=(Bc, S // BS),
        in_specs=[
            pl.BlockSpec((1, BS, Nc), lambda b, i: (b, i, 0)),
        ],
        out_specs=pl.BlockSpec((1, BS, nsample), lambda b, i: (b, i, 0)),
        out_shape=jax.ShapeDtypeStruct((Bc, S, nsample), jnp.int32),
    )(d2)


def _sa_group(npoint, radius, nsample, xyz_t):
    fps_idx = _fps(xyz_t, npoint)
    new_xyz = _index_points(xyz_t, fps_idx)
    idx = _query_ball_pallas(radius, nsample, xyz_t, new_xyz)
    return new_xyz, idx


def _sa_apply(ps, xyz_t, points, new_xyz, idx):
    points_t = jnp.transpose(points, (0, 2, 1))
    grouped_xyz = _index_points(xyz_t, idx) - new_xyz[:, :, None, :]
    grouped_points = _index_points(points_t, idx)
    new_points = jnp.concatenate([grouped_xyz, grouped_points], axis=-1)
    x = jnp.transpose(new_points, (0, 3, 2, 1))
    for layer in ps:
        x = jnp.einsum('oc,bcns->bons', layer["W"], x) + layer["b"][None, :, None, None]
        x = jax.nn.relu(_batchnorm(x, (0, 2, 3)))
    x = jnp.max(x, axis=2)
    return jnp.transpose(new_xyz, (0, 2, 1)), x


def _interp_kernel(d_ref, p2_ref, o_ref, *, S2):
    d = d_ref[0]
    iota = jax.lax.broadcasted_iota(jnp.int32, d.shape, 1)
    sels, ds = [], []
    for _ in range(3):
        m = jnp.min(d, axis=1, keepdims=True)
        amin = jnp.min(jnp.where(d == m, iota, S2), axis=1, keepdims=True)
        sel = iota == amin
        sels.append(sel)
        ds.append(m)
        d = jnp.where(sel, jnp.float32(jnp.inf), d)
    recips = [1.0 / (dj + 1e-8) for dj in ds]
    norm = recips[0] + recips[1] + recips[2]
    W = sum((r / norm) * s.astype(jnp.float32) for r, s in zip(recips, sels))
    o_ref[0] = jnp.dot(W, p2_ref[0], preferred_element_type=jnp.float32,
                       precision=jax.lax.Precision.HIGHEST)


def _fp_interp_pallas(xyz1_t, xyz2_t, points2_t):
    Bc, S1, _ = xyz1_t.shape
    S2 = xyz2_t.shape[1]
    C = points2_t.shape[2]
    d = _square_distance(xyz1_t, xyz2_t)
    BS = min(S1, 512)
    kfn = functools.partial(_interp_kernel, S2=S2)
    return pl.pallas_call(
        kfn,
        grid=(Bc, S1 // BS),
        in_specs=[
            pl.BlockSpec((1, BS, S2), lambda b, i: (b, i, 0)),
            pl.BlockSpec((1, S2, C), lambda b, i: (b, 0, 0)),
        ],
        out_specs=pl.BlockSpec((1, BS, C), lambda b, i: (b, i, 0)),
        out_shape=jax.ShapeDtypeStruct((Bc, S1, C), jnp.float32),
    )(d, points2_t)


def _fp_layer(ps, xyz1, xyz2, points1, points2):
    xyz1_t = jnp.transpose(xyz1, (0, 2, 1))
    xyz2_t = jnp.transpose(xyz2, (0, 2, 1))
    points2_t = jnp.transpose(points2, (0, 2, 1))
    interpolated = _fp_interp_pallas(xyz1_t, xyz2_t, points2_t)
    if points1 is not None:
        points1_t = jnp.transpose(points1, (0, 2, 1))
        new_points = jnp.concatenate([points1_t, interpolated], axis=-1)
    else:
        new_points = interpolated
    x = jnp.transpose(new_points, (0, 2, 1))
    for layer in ps:
        x = jnp.einsum('oc,bcn->bon', layer["W"], x) + layer["b"][None, :, None]
        x = jax.nn.relu(_batchnorm(x, (0, 2)))
    return x


def _enet_features(ps, imgs):
    dn = ('NCHW', 'OIHW', 'NCHW')
    x = jax.lax.conv_general_dilated(imgs, ps["W1"], (4, 4), 'VALID', dimension_numbers=dn)
    x = jax.nn.relu(x + ps["b1"][None, :, None, None])
    x = jax.lax.conv_general_dilated(x, ps["W2"], (2, 2), 'VALID', dimension_numbers=dn)
    x = jax.nn.relu(x + ps["b2"][None, :, None, None])
    return x


def _projection(ft, ind3d, ind2d, num_points):
    C = ft.shape[0]
    flat = ft.reshape(C, -1)
    gathered = flat[:, ind2d]
    return jnp.zeros((C, num_points), ft.dtype).at[:, ind3d].set(gathered)


def _head_kernel(x_ref, w1_ref, b1_ref, w2_ref, b2_ref, o_ref):
    x = x_ref[...]
    y = jnp.dot(w1_ref[...], x, preferred_element_type=jnp.float32)
    y = y + b1_ref[...].reshape(-1, 1)
    mean = jnp.mean(y, axis=1, keepdims=True)
    var = jnp.mean((y - mean) ** 2, axis=1, keepdims=True)
    y = jax.nn.relu((y - mean) / jnp.sqrt(var + 1e-5))
    z = jnp.dot(w2_ref[...], y, preferred_element_type=jnp.float32)
    o_ref[...] = z + b2_ref[...].reshape(-1, 1)


def _head(params, l0_points):
    Bc, C, Nc = l0_points.shape
    x = jnp.transpose(l0_points, (1, 0, 2)).reshape(C, Bc * Nc)
    out = pl.pallas_call(
        _head_kernel,
        out_shape=jax.ShapeDtypeStruct((_NUM_CLASSES, Bc * Nc), jnp.float32),
    )(x, params["conv1"]["W"], params["conv1"]["b"],
      params["conv2"]["W"], params["conv2"]["b"])
    return jnp.transpose(out.reshape(_NUM_CLASSES, Bc, Nc), (1, 2, 0))


def kernel(params, xyz, points, image, projection_indices_3d, projection_indices_2d):
    num_points = xyz.shape[2]
    feats = []
    for i in range(_B):
        ft = _enet_features(params["enet"], image[i])
        proj = [_projection(ft[j], projection_indices_3d[i, j],
                            projection_indices_2d[i, j], num_points)
                for j in range(_NUM_IMAGES)]
        imageft = jnp.stack(proj, axis=2)
        imageft = jnp.max(imageft, axis=2)
        feats.append(imageft)
    image_features = jnp.stack(feats, axis=0)
    xyz_t = jnp.transpose(xyz, (0, 2, 1))
    nx1, idx1 = _sa_group(1024, 0.1, 32, xyz_t)
    l1_xyz, l1_points = _sa_apply(params["sa1_geo"], xyz_t, points, nx1, idx1)
    _, l1_points_f = _sa_apply(params["sa1_feat"], xyz_t, image_features, nx1, idx1)
    l1_xyz_t = nx1
    nx2, idx2 = _sa_group(256, 0.2, 32, l1_xyz_t)
    l2_xyz, l2_points = _sa_apply(params["sa2_geo"], l1_xyz_t, l1_points, nx2, idx2)
    _, l2_points_f = _sa_apply(params["sa2_feat"], l1_xyz_t, l1_points_f, nx2, idx2)
    l1_xyz_f = l1_xyz
    l2_points = jnp.concatenate([l2_points, l2_points_f], axis=1)
    l2_xyz_t = nx2
    nx3, idx3 = _sa_group(64, 0.4, 32, l2_xyz_t)
    l3_xyz, l3_points = _sa_apply(params["sa3"], l2_xyz_t, l2_points, nx3, idx3)
    l3_xyz_t = nx3
    nx4, idx4 = _sa_group(16, 0.8, 32, l3_xyz_t)
    l4_xyz, l4_points = _sa_apply(params["sa4"], l3_xyz_t, l3_points, nx4, idx4)
    l3_points = _fp_layer(params["fp4"], l3_xyz, l4_xyz, l3_points, l4_points)
    l2_points = _fp_layer(params["fp3"], l2_xyz, l3_xyz, l2_points, l3_points)
    l1_points = _fp_layer(params["fp2"], l1_xyz, l2_xyz, l1_points, l2_points)
    l0_points = _fp_layer(params["fp1"], xyz, l1_xyz, points, l1_points)
    return _head(params, l0_points)


# speedup vs baseline: 3.3543x; 2.3002x over previous
import functools

import jax
import jax.numpy as jnp
import numpy as np
from jax.experimental import pallas as pl
from jax.experimental.pallas import tpu as pltpu

_B = 2
_N = 8192
_NUM_IMAGES = 3
_IMG_H, _IMG_W = 256, 320
_FEAT_H, _FEAT_W = 32, 40
_P_PROJ = 4096
_NUM_CLASSES = 21


def _batchnorm(x, axes):
    mean = jnp.mean(x, axis=axes, keepdims=True)
    var = jnp.var(x, axis=axes, keepdims=True)
    return (x - mean) / jnp.sqrt(var + 1e-5)


def _square_distance(a, b):
    return (jnp.sum(a * a, -1)[:, :, None] + jnp.sum(b * b, -1)[:, None, :]
            - 2.0 * jnp.einsum('bnc,bmc->bnm', a, b))


def _index_points(pts, idx):
    return jax.vmap(lambda p, i: p[i])(pts, idx)


def _fps_kernel(xt_ref, o_ref, *, npoint, Nc):
    xyz = xt_ref[0]
    iota = jax.lax.broadcasted_iota(jnp.int32, (1, Nc), 1)
    iota3 = jax.lax.broadcasted_iota(jnp.int32, (3, Nc), 1)

    def body(i, state):
        distance, farthest, picked_at = state
        picked_at = jnp.where(iota == farthest, i, picked_at)
        sel = iota3 == farthest
        centroid = jnp.sum(jnp.where(sel, xyz, 0.0), axis=1, keepdims=True)
        diff = xyz - centroid
        dist = jnp.sum(diff * diff, axis=0, keepdims=True)
        distance = jnp.minimum(distance, dist)
        m = jnp.max(distance, axis=1, keepdims=True)
        farthest = jnp.min(jnp.where(distance == m, iota, Nc), axis=1,
                           keepdims=True)
        return distance, farthest, picked_at

    _, _, picked = jax.lax.fori_loop(
        0, npoint, body,
        (jnp.full((1, Nc), 1e10, jnp.float32),
         jnp.zeros((1, 1), jnp.int32),
         jnp.full((1, Nc), Nc, jnp.int32)))
    o_ref[0] = picked


def _fps(xyz, npoint):
    xyz = jax.lax.stop_gradient(xyz)
    Bc, Nc, _ = xyz.shape
    xt = jnp.transpose(xyz, (0, 2, 1))
    picked = pl.pallas_call(
        functools.partial(_fps_kernel, npoint=npoint, Nc=Nc),
        grid=(Bc,),
        in_specs=[pl.BlockSpec((1, 3, Nc), lambda b: (b, 0, 0))],
        out_specs=pl.BlockSpec((1, 1, Nc), lambda b: (b, 0, 0)),
        out_shape=jax.ShapeDtypeStruct((Bc, 1, Nc), jnp.int32),
    )(xt)[:, 0]
    return jnp.argsort(picked, axis=1)[:, :npoint].astype(jnp.int32)


def _bq_kernel(d2_ref, o_ref, *, r2, K, Nc):
    d2 = d2_ref[0]
    iota = jax.lax.broadcasted_iota(jnp.int32, d2.shape, 1)
    m = jnp.where(d2 <= r2, iota, Nc)
    cols = []
    for _ in range(K):
        cur = jnp.min(m, axis=1, keepdims=True)
        cols.append(cur)
        m = jnp.where(m == cur, Nc, m)
    idx = jnp.concatenate(cols, axis=1)
    first = idx[:, 0:1]
    o_ref[0] = jnp.where(idx == Nc, jnp.broadcast_to(first, idx.shape), idx)


def _query_ball_pallas(radius, nsample, xyz_t, new_xyz):
    Bc, S, _ = new_xyz.shape
    Nc = xyz_t.shape[1]
    d2 = jax.lax.stop_gradient(_square_distance(new_xyz, xyz_t))
    BS = min(S, 256)
    kfn = functools.partial(_bq_kernel, r2=radius ** 2, K=nsample, Nc=Nc)
    return pl.pallas_call(
        kfn,
        grid=(Bc, S // BS),
        in_specs=[
            pl.BlockSpec((1, BS, Nc), lambda b, i: (b, i, 0)),
        ],
        out_specs=pl.BlockSpec((1, BS, nsample), lambda b, i: (b, i, 0)),
        out_shape=jax.ShapeDtypeStruct((Bc, S, nsample), jnp.int32),
    )(d2)


def _sa_group(npoint, radius, nsample, xyz_t):
    fps_idx = _fps(xyz_t, npoint)
    new_xyz = _index_points(xyz_t, fps_idx)
    idx = _query_ball_pallas(radius, nsample, xyz_t, new_xyz)
    return new_xyz, idx


def _sa_apply(ps, xyz_t, points, new_xyz, idx):
    points_t = jnp.transpose(points, (0, 2, 1))
    grouped_xyz = _index_points(xyz_t, idx) - new_xyz[:, :, None, :]
    grouped_points = _index_points(points_t, idx)
    new_points = jnp.concatenate([grouped_xyz, grouped_points], axis=-1)
    x = jnp.transpose(new_points, (0, 3, 2, 1))
    for layer in ps:
        x = jnp.einsum('oc,bcns->bons', layer["W"], x) + layer["b"][None, :, None, None]
        x = jax.nn.relu(_batchnorm(x, (0, 2, 3)))
    x = jnp.max(x, axis=2)
    return jnp.transpose(new_xyz, (0, 2, 1)), x


def _interp_kernel(d_ref, p2_ref, o_ref, *, S2):
    d = d_ref[0]
    iota = jax.lax.broadcasted_iota(jnp.int32, d.shape, 1)
    sels, ds = [], []
    for _ in range(3):
        m = jnp.min(d, axis=1, keepdims=True)
        amin = jnp.min(jnp.where(d == m, iota, S2), axis=1, keepdims=True)
        sel = iota == amin
        sels.append(sel)
        ds.append(m)
        d = jnp.where(sel, jnp.float32(jnp.inf), d)
    recips = [1.0 / (dj + 1e-8) for dj in ds]
    norm = recips[0] + recips[1] + recips[2]
    W = sum((r / norm) * s.astype(jnp.float32) for r, s in zip(recips, sels))
    o_ref[0] = jnp.dot(W, p2_ref[0], preferred_element_type=jnp.float32,
                       precision=jax.lax.Precision.HIGHEST)


def _fp_interp_pallas(xyz1_t, xyz2_t, points2_t):
    Bc, S1, _ = xyz1_t.shape
    S2 = xyz2_t.shape[1]
    C = points2_t.shape[2]
    d = _square_distance(xyz1_t, xyz2_t)
    BS = min(S1, 512)
    kfn = functools.partial(_interp_kernel, S2=S2)
    return pl.pallas_call(
        kfn,
        grid=(Bc, S1 // BS),
        in_specs=[
            pl.BlockSpec((1, BS, S2), lambda b, i: (b, i, 0)),
            pl.BlockSpec((1, S2, C), lambda b, i: (b, 0, 0)),
        ],
        out_specs=pl.BlockSpec((1, BS, C), lambda b, i: (b, i, 0)),
        out_shape=jax.ShapeDtypeStruct((Bc, S1, C), jnp.float32),
    )(d, points2_t)


def _fp_layer(ps, xyz1, xyz2, points1, points2):
    xyz1_t = jnp.transpose(xyz1, (0, 2, 1))
    xyz2_t = jnp.transpose(xyz2, (0, 2, 1))
    points2_t = jnp.transpose(points2, (0, 2, 1))
    interpolated = _fp_interp_pallas(xyz1_t, xyz2_t, points2_t)
    if points1 is not None:
        points1_t = jnp.transpose(points1, (0, 2, 1))
        new_points = jnp.concatenate([points1_t, interpolated], axis=-1)
    else:
        new_points = interpolated
    x = jnp.transpose(new_points, (0, 2, 1))
    for layer in ps:
        x = jnp.einsum('oc,bcn->bon', layer["W"], x) + layer["b"][None, :, None]
        x = jax.nn.relu(_batchnorm(x, (0, 2)))
    return x


def _enet_features(ps, imgs):
    dn = ('NCHW', 'OIHW', 'NCHW')
    x = jax.lax.conv_general_dilated(imgs, ps["W1"], (4, 4), 'VALID', dimension_numbers=dn)
    x = jax.nn.relu(x + ps["b1"][None, :, None, None])
    x = jax.lax.conv_general_dilated(x, ps["W2"], (2, 2), 'VALID', dimension_numbers=dn)
    x = jax.nn.relu(x + ps["b2"][None, :, None, None])
    return x


def _projection(ft, ind3d, ind2d, num_points):
    C = ft.shape[0]
    flat = ft.reshape(C, -1)
    gathered = flat[:, ind2d]
    return jnp.zeros((C, num_points), ft.dtype).at[:, ind3d].set(gathered)


def _head_kernel(x_ref, w1_ref, b1_ref, w2_ref, b2_ref, o_ref):
    x = x_ref[...]
    y = jnp.dot(w1_ref[...], x, preferred_element_type=jnp.float32)
    y = y + b1_ref[...].reshape(-1, 1)
    mean = jnp.mean(y, axis=1, keepdims=True)
    var = jnp.mean((y - mean) ** 2, axis=1, keepdims=True)
    y = jax.nn.relu((y - mean) / jnp.sqrt(var + 1e-5))
    z = jnp.dot(w2_ref[...], y, preferred_element_type=jnp.float32)
    o_ref[...] = z + b2_ref[...].reshape(-1, 1)


def _head(params, l0_points):
    Bc, C, Nc = l0_points.shape
    x = jnp.transpose(l0_points, (1, 0, 2)).reshape(C, Bc * Nc)
    out = pl.pallas_call(
        _head_kernel,
        out_shape=jax.ShapeDtypeStruct((_NUM_CLASSES, Bc * Nc), jnp.float32),
    )(x, params["conv1"]["W"], params["conv1"]["b"],
      params["conv2"]["W"], params["conv2"]["b"])
    return jnp.transpose(out.reshape(_NUM_CLASSES, Bc, Nc), (1, 2, 0))


def kernel(params, xyz, points, image, projection_indices_3d, projection_indices_2d):
    num_points = xyz.shape[2]
    feats = []
    for i in range(_B):
        ft = _enet_features(params["enet"], image[i])
        proj = [_projection(ft[j], projection_indices_3d[i, j],
                            projection_indices_2d[i, j], num_points)
                for j in range(_NUM_IMAGES)]
        imageft = jnp.stack(proj, axis=2)
        imageft = jnp.max(imageft, axis=2)
        feats.append(imageft)
    image_features = jnp.stack(feats, axis=0)
    xyz_t = jnp.transpose(xyz, (0, 2, 1))
    nx1, idx1 = _sa_group(1024, 0.1, 32, xyz_t)
    l1_xyz, l1_points = _sa_apply(params["sa1_geo"], xyz_t, points, nx1, idx1)
    _, l1_points_f = _sa_apply(params["sa1_feat"], xyz_t, image_features, nx1, idx1)
    l1_xyz_t = nx1
    nx2, idx2 = _sa_group(256, 0.2, 32, l1_xyz_t)
    l2_xyz, l2_points = _sa_apply(params["sa2_geo"], l1_xyz_t, l1_points, nx2, idx2)
    _, l2_points_f = _sa_apply(params["sa2_feat"], l1_xyz_t, l1_points_f, nx2, idx2)
    l1_xyz_f = l1_xyz
    l2_points = jnp.concatenate([l2_points, l2_points_f], axis=1)
    l2_xyz_t = nx2
    nx3, idx3 = _sa_group(64, 0.4, 32, l2_xyz_t)
    l3_xyz, l3_points = _sa_apply(params["sa3"], l2_xyz_t, l2_points, nx3, idx3)
    l3_xyz_t = nx3
    nx4, idx4 = _sa_group(16, 0.8, 32, l3_xyz_t)
    l4_xyz, l4_points = _sa_apply(params["sa4"], l3_xyz_t, l3_points, nx4, idx4)
    l3_points = _fp_layer(params["fp4"], l3_xyz, l4_xyz, l3_points, l4_points)
    l2_points = _fp_layer(params["fp3"], l2_xyz, l3_xyz, l2_points, l3_points)
    l1_points = _fp_layer(params["fp2"], l1_xyz, l2_xyz, l1_points, l2_points)
    l0_points = _fp_layer(params["fp1"], xyz, l1_xyz, points, l1_points)
    return _head(params, l0_points)
